# Initial kernel scaffold; baseline (speedup 1.0000x reference)
import jax
import jax.numpy as jnp
from jax.experimental import pallas as pl


def kernel(x, weight):
    raise NotImplementedError("write your pallas kernel here")


# speedup vs baseline: 1.5726x; 1.5726x over previous
import jax
import jax.numpy as jnp
from jax import lax
from jax.experimental import pallas as pl
from jax.experimental.pallas import tpu as pltpu
from jax.experimental.pallas import tpu_sc as plsc

_EMB = 32
_NC = 2
_NS = 16
_NW = _NC * _NS

_B = 16384 * 26
_BPW = _B // _NW
_C = 3328
_NCHUNK = _BPW // _C


def _emb_body(x_hbm, w_hbm, out_hbm, idx_v, rows_v, sem):
    wid = lax.axis_index("s") * _NC + lax.axis_index("c")
    wbase = wid * _BPW
    for i in range(_NCHUNK):
        base = wbase + i * _C
        pltpu.sync_copy(x_hbm.at[pl.ds(base, _C)], idx_v)
        pltpu.async_copy(w_hbm.at[idx_v], rows_v, sem).wait()
        pltpu.sync_copy(rows_v, out_hbm.at[pl.ds(base, _C)])


def kernel(x, weight):
    bb, ff = x.shape
    xf = x.reshape(bb * ff).astype(jnp.int32)
    run = pl.kernel(
        _emb_body,
        out_type=jax.ShapeDtypeStruct((bb * ff, _EMB), jnp.float32),
        mesh=plsc.VectorSubcoreMesh(core_axis_name="c", subcore_axis_name="s"),
        compiler_params=pltpu.CompilerParams(use_tc_tiling_on_sc=False),
        scratch_types=[
            pltpu.VMEM((_C,), jnp.int32),
            pltpu.VMEM((_C, _EMB), jnp.float32),
            pltpu.SemaphoreType.DMA,
        ],
    )
    out = run(xf, weight)
    return out.reshape(bb, ff, _EMB)
